# baseline (device time: 431977 ns/iter reference)
import jax
import jax.numpy as jnp
from jax import lax
from jax.experimental import pallas as pl
from jax.experimental.pallas import tpu as pltpu

N_DEV = 4
SQ = 2048
SKV_SHARD = 2048
HQ = 8
DH = 128
DM = HQ * DH
BQ = 128
BLK = 64
SCALE = 0.08838834764831843
LOG2E = 1.4426950408889634


def _qproj_body(x_ref, wq_ref, q_ref):
    for h in range(HQ):
        qh = jnp.dot(x_ref[...], wq_ref[:, h * DH:(h + 1) * DH],
                     preferred_element_type=jnp.float32)
        q_ref[h] = qh.astype(jnp.bfloat16)


def _body(q_ref, k_ref, v_ref, wo_ref, out_ref,
          kg_ref, vg_ref, num_ref, den_ref, ksend, krecv, vsend, vrecv):
    me = lax.axis_index("i")
    left = lax.rem(me + N_DEV - 1, N_DEV)
    right = lax.rem(me + 1, N_DEV)

    barrier_sem = pltpu.get_barrier_semaphore()
    pl.semaphore_signal(barrier_sem, inc=1, device_id=(left,),
                        device_id_type=pl.DeviceIdType.MESH)
    pl.semaphore_signal(barrier_sem, inc=1, device_id=(right,),
                        device_id_type=pl.DeviceIdType.MESH)
    pl.semaphore_wait(barrier_sem, 2)

    def make_send(h):
        k_src = k_ref if h == 0 else kg_ref.at[h - 1]
        v_src = v_ref if h == 0 else vg_ref.at[h - 1]
        k_send = pltpu.make_async_remote_copy(
            src_ref=k_src, dst_ref=kg_ref.at[h],
            send_sem=ksend.at[h], recv_sem=krecv.at[h],
            device_id=(right,), device_id_type=pl.DeviceIdType.MESH)
        v_send = pltpu.make_async_remote_copy(
            src_ref=v_src, dst_ref=vg_ref.at[h],
            send_sem=vsend.at[h], recv_sem=vrecv.at[h],
            device_id=(right,), device_id_type=pl.DeviceIdType.MESH)
        return k_send, v_send

    def process(get_k, get_v, org, init):

        def qb_step(qb, carry):
            rb = (me * SQ + qb * BQ
                  + lax.broadcasted_iota(jnp.int32, (BQ, 1), 0)) // BLK
            cb = (org * SKV_SHARD
                  + lax.broadcasted_iota(jnp.int32, (1, SKV_SHARD), 1)) // BLK
            s3 = lax.rem(rb, 3) + lax.rem(cb, 3)
            keep = (rb == cb) | (cb == 0) | (s3 == 0) | (s3 == 3)
            keepf = keep.astype(jnp.bfloat16)
            rows = pl.ds(qb * BQ, BQ)

            ones = jnp.ones((SKV_SHARD, DH), jnp.bfloat16)

            def h_step(h, hcarry):
                q = q_ref[h, rows, :]
                s = lax.dot_general(q, get_k(h), (((1,), (1,)), ((), ())),
                                    preferred_element_type=jnp.float32)
                w = lax.exp2(s.astype(jnp.bfloat16)) * keepf
                d = jnp.dot(w, ones, preferred_element_type=jnp.float32)
                n = jnp.dot(w, get_v(h),
                            preferred_element_type=jnp.float32)
                if init:
                    num_ref[h, rows, :] = n
                    den_ref[h, rows, :] = d
                else:
                    num_ref[h, rows, :] += n
                    den_ref[h, rows, :] += d
                return hcarry

            lax.fori_loop(0, HQ, h_step, 0)
            return carry

        lax.fori_loop(0, SQ // BQ, qb_step, 0)

    sends = [make_send(0)]
    sends[0][0].start()
    sends[0][1].start()
    process(lambda h: k_ref[h], lambda h: v_ref[h], me, init=True)

    for h in range(N_DEV - 1):
        k_recv = pltpu.make_async_remote_copy(
            src_ref=kg_ref.at[h], dst_ref=kg_ref.at[h],
            send_sem=ksend.at[h], recv_sem=krecv.at[h],
            device_id=(left,), device_id_type=pl.DeviceIdType.MESH)
        v_recv = pltpu.make_async_remote_copy(
            src_ref=vg_ref.at[h], dst_ref=vg_ref.at[h],
            send_sem=vsend.at[h], recv_sem=vrecv.at[h],
            device_id=(left,), device_id_type=pl.DeviceIdType.MESH)
        k_recv.wait_recv()
        v_recv.wait_recv()
        if h < N_DEV - 2:
            nxt = make_send(h + 1)
            nxt[0].start()
            nxt[1].start()
            sends.append(nxt)
        org = lax.rem(me - h - 1 + N_DEV, N_DEV)
        process(lambda hh: kg_ref[h, hh], lambda hh: vg_ref[h, hh],
                org, init=False)

    for k_send, v_send in sends:
        k_send.wait_send()
        v_send.wait_send()

    RCH = 512
    for r0 in range(0, SQ, RCH):
        acc = jnp.zeros((RCH, DM), jnp.float32)
        for h in range(HQ):
            ctx = (num_ref[h, r0:r0 + RCH, :]
                   / den_ref[h, r0:r0 + RCH, :]).astype(jnp.bfloat16)
            acc = acc + jnp.dot(ctx, wo_ref[h * DH:(h + 1) * DH, :],
                                preferred_element_type=jnp.float32)
        out_ref[r0:r0 + RCH, :] = acc.astype(jnp.bfloat16)


def kernel(x, Wq, K_ext, V_ext, Wo):
    x2 = x[0].astype(jnp.bfloat16)
    wq = (Wq * (SCALE * LOG2E)).astype(jnp.bfloat16)
    kt = K_ext[0].transpose(1, 0, 2).astype(jnp.bfloat16)
    vt = V_ext[0].transpose(1, 0, 2).astype(jnp.bfloat16)
    wo = Wo.astype(jnp.bfloat16)

    q = pl.pallas_call(
        _qproj_body,
        out_shape=jax.ShapeDtypeStruct((HQ, SQ, DH), jnp.bfloat16),
        in_specs=[pl.BlockSpec(memory_space=pltpu.VMEM)] * 2,
        out_specs=pl.BlockSpec(memory_space=pltpu.VMEM),
        compiler_params=pltpu.CompilerParams(
            vmem_limit_bytes=32 * 1024 * 1024),
    )(x2, wq)

    out = pl.pallas_call(
        _body,
        out_shape=jax.ShapeDtypeStruct((SQ, DM), jnp.bfloat16),
        in_specs=[pl.BlockSpec(memory_space=pltpu.VMEM)] * 4,
        out_specs=pl.BlockSpec(memory_space=pltpu.VMEM),
        scratch_shapes=[
            pltpu.VMEM((N_DEV - 1, HQ, SKV_SHARD, DH), jnp.bfloat16),
            pltpu.VMEM((N_DEV - 1, HQ, SKV_SHARD, DH), jnp.bfloat16),
            pltpu.VMEM((HQ, SQ, DH), jnp.float32),
            pltpu.VMEM((HQ, SQ, DH), jnp.float32),
            pltpu.SemaphoreType.DMA((N_DEV - 1,)),
            pltpu.SemaphoreType.DMA((N_DEV - 1,)),
            pltpu.SemaphoreType.DMA((N_DEV - 1,)),
            pltpu.SemaphoreType.DMA((N_DEV - 1,)),
        ],
        compiler_params=pltpu.CompilerParams(
            collective_id=0, vmem_limit_bytes=63 * 1024 * 1024),
    )(q, kt, vt, wo)
    return out[None].astype(jnp.float32)
